# baseline (speedup 1.0000x reference)
import functools

import jax
import jax.numpy as jnp
from jax import lax
from jax.experimental import pallas as pl
from jax.experimental.pallas import tpu as pltpu
from jax.experimental.pallas import tpu_sc as plsc

IDXW = 128


def _build(N, D, V):
    info = plsc.get_sparse_core_info()
    NC, NS = info.num_cores, info.num_subcores
    NW = NC * NS
    assert N % (NW * IDXW) == 0
    b_per_w = N // NW
    K = 4
    CH = K * IDXW
    n_ch = b_per_w // CH

    mesh = plsc.VectorSubcoreMesh(core_axis_name="c", subcore_axis_name="s")

    @functools.partial(
        pl.kernel,
        mesh=mesh,
        compiler_params=pltpu.CompilerParams(use_tc_tiling_on_sc=False),
        out_type=jax.ShapeDtypeStruct((N, D), jnp.float32),
        scratch_types=[
            pltpu.VMEM((K, IDXW), jnp.int32),
            pltpu.VMEM((CH, D), jnp.float32),
            pltpu.SemaphoreType.DMA,
        ],
    )
    def emb_kernel(idx_hbm, tbl_hbm, out_hbm, idx_v, rows_v, sem):
        wid = lax.axis_index("s") * NC + lax.axis_index("c")
        row0 = wid * (b_per_w // IDXW)

        def body(c, carry):
            r = row0 + c * K
            pltpu.sync_copy(idx_hbm.at[pl.ds(r, K)], idx_v)
            copies = [
                pltpu.async_copy(
                    tbl_hbm.at[idx_v.at[j]],
                    rows_v.at[pl.ds(j * IDXW, IDXW)],
                    sem,
                )
                for j in range(K)
            ]
            for cp in copies:
                cp.wait()
            pltpu.sync_copy(rows_v, out_hbm.at[pl.ds(r * IDXW, CH)])
            return carry

        lax.fori_loop(0, n_ch, body, 0)

    return emb_kernel


def kernel(x, table):
    B, S = x.shape
    V, D = table.shape
    N = B * S
    idx2d = x.reshape(N // IDXW, IDXW).astype(jnp.int32)
    out = _build(N, D, V)(idx2d, table)
    return out.reshape(B, S, D)


# speedup vs baseline: 1.0337x; 1.0337x over previous
import functools

import jax
import jax.numpy as jnp
from jax import lax
from jax.experimental import pallas as pl
from jax.experimental.pallas import tpu as pltpu
from jax.experimental.pallas import tpu_sc as plsc

IDXW = 128
K = 5
CH = K * IDXW


def _build(N, D, V):
    info = plsc.get_sparse_core_info()
    NC, NS = info.num_cores, info.num_subcores
    NW = NC * NS
    assert N % (NW * CH) == 0
    b_per_w = N // NW
    n_ch = b_per_w // CH
    assert n_ch % 2 == 0 and n_ch >= 4

    mesh = plsc.VectorSubcoreMesh(core_axis_name="c", subcore_axis_name="s")

    @functools.partial(
        pl.kernel,
        mesh=mesh,
        compiler_params=pltpu.CompilerParams(use_tc_tiling_on_sc=False),
        out_type=jax.ShapeDtypeStruct((N, D), jnp.float32),
        scratch_types=[
            pltpu.VMEM((2, K, IDXW), jnp.int32),
            pltpu.VMEM((2, CH, D), jnp.float32),
            pltpu.SemaphoreType.DMA,
            pltpu.SemaphoreType.DMA,
        ],
    )
    def emb_kernel(idx_hbm, tbl_hbm, out_hbm, idx_v, rows_v, sg0, sg1):
        wid = lax.axis_index("s") * NC + lax.axis_index("c")
        row0 = wid * (b_per_w // IDXW)
        base = wid * b_per_w
        sg = (sg0, sg1)

        def load_idx(b, c):
            pltpu.sync_copy(idx_hbm.at[pl.ds(row0 + c * K, K)], idx_v.at[b])

        def gathers(b):
            return [
                pltpu.make_async_copy(
                    tbl_hbm.at[idx_v.at[b].at[j]],
                    rows_v.at[b].at[pl.ds(j * IDXW, IDXW)],
                    sg[b],
                )
                for j in range(K)
            ]

        def fire_gathers(b):
            for cp in gathers(b):
                cp.start()

        def wait_gathers(b):
            for cp in gathers(b):
                cp.wait()

        def write_out(b, c):
            pltpu.sync_copy(rows_v.at[b], out_hbm.at[pl.ds(base + c * CH, CH)])

        for b in (0, 1):
            load_idx(b, b)
            fire_gathers(b)

        @pl.loop(0, n_ch - 2, step=2)
        def _steady(g):
            for b in (0, 1):
                c = g + b
                wait_gathers(b)
                write_out(b, c)
                load_idx(b, c + 2)
                fire_gathers(b)

        for b in (0, 1):
            wait_gathers(b)
            write_out(b, n_ch - 2 + b)

    return emb_kernel


def kernel(x, table):
    B, S = x.shape
    V, D = table.shape
    N = B * S
    idx2d = x.reshape(N // IDXW, IDXW).astype(jnp.int32)
    out = _build(N, D, V)(idx2d, table)
    return out.reshape(B, S, D)


# speedup vs baseline: 1.3779x; 1.3330x over previous
import functools

import jax
import jax.numpy as jnp
from jax import lax
from jax.experimental import pallas as pl
from jax.experimental.pallas import tpu as pltpu
from jax.experimental.pallas import tpu_sc as plsc

IDXW = 128
K = 5
CH = K * IDXW


def _build(N, D, V):
    info = plsc.get_sparse_core_info()
    NC, NS = info.num_cores, info.num_subcores
    NW = NC * NS
    assert N % (NW * CH) == 0
    b_per_w = N // NW
    n_ch = b_per_w // CH
    assert n_ch % 2 == 0 and n_ch >= 4

    mesh = plsc.VectorSubcoreMesh(core_axis_name="c", subcore_axis_name="s")

    @functools.partial(
        pl.kernel,
        mesh=mesh,
        compiler_params=pltpu.CompilerParams(use_tc_tiling_on_sc=False),
        out_type=jax.ShapeDtypeStruct((N, 128), jnp.float32),
        scratch_types=[
            pltpu.VMEM((2, K, IDXW), jnp.int32),
            pltpu.VMEM((2, CH, D), jnp.float32),
            pltpu.SemaphoreType.DMA,
            pltpu.SemaphoreType.DMA,
        ],
    )
    def emb_kernel(idx_hbm, tbl_hbm, out_hbm, idx_v, rows_v, sg0, sg1):
        wid = lax.axis_index("s") * NC + lax.axis_index("c")
        row0 = wid * (b_per_w // IDXW)
        base = wid * b_per_w
        sg = (sg0, sg1)

        def load_idx(b, c):
            pltpu.sync_copy(idx_hbm.at[pl.ds(row0 + c * K, K)], idx_v.at[b])

        def gathers(b):
            return [
                pltpu.make_async_copy(
                    tbl_hbm.at[idx_v.at[b].at[j]],
                    rows_v.at[b].at[pl.ds(j * IDXW, IDXW)],
                    sg[b],
                )
                for j in range(K)
            ]

        def fire_gathers(b):
            for cp in gathers(b):
                cp.start()

        def wait_gathers(b):
            for cp in gathers(b):
                cp.wait()

        def write_out(b, c):
            pltpu.sync_copy(
                rows_v.at[b],
                out_hbm.at[pl.ds(base + c * CH, CH), pl.ds(0, D)],
            )

        for b in (0, 1):
            load_idx(b, b)
            fire_gathers(b)

        @pl.loop(0, n_ch - 2, step=2)
        def _steady(g):
            for b in (0, 1):
                c = g + b
                wait_gathers(b)
                write_out(b, c)
                load_idx(b, c + 2)
                fire_gathers(b)

        for b in (0, 1):
            wait_gathers(b)
            write_out(b, n_ch - 2 + b)

    return emb_kernel


def kernel(x, table):
    B, S = x.shape
    V, D = table.shape
    N = B * S
    idx2d = x.reshape(N // IDXW, IDXW).astype(jnp.int32)
    out = _build(N, D, V)(idx2d, table)
    return out[:, :D].reshape(B, S, D)
